# baseline (speedup 1.0000x reference)
import functools

import jax
import jax.numpy as jnp
from jax import lax
from jax.experimental import pallas as pl
from jax.experimental.pallas import tpu as pltpu, tpu_sc as plsc

B, C, T, H, W = 8, 64, 32, 56, 56
NC, NS = 2, 16
NW = NC * NS
SLABS = B * T
SPW = SLABS // NW
ROWS = SLABS * H
NSEM = 4


@functools.partial(
    pl.kernel,
    out_type=jax.ShapeDtypeStruct((ROWS, W, C), jnp.float32),
    mesh=plsc.VectorSubcoreMesh(core_axis_name="c", subcore_axis_name="s"),
    scratch_types=[
        pltpu.VMEM((T + 16,), jnp.int32),
    ] + [pltpu.SemaphoreType.DMA] * NSEM,
)
def _sc_shuffle(x_hbm, idx_hbm, out_hbm, idxs_v, *sems):
    wid = lax.axis_index("s") * NC + lax.axis_index("c")
    pltpu.sync_copy(idx_hbm, idxs_v.at[pl.ds(0, T)])

    def rows(s):
        d = wid * SPW + s
        b = lax.shift_right_logical(d, 5)
        j = lax.bitwise_and(d, T - 1)
        pj = idxs_v[pl.ds(j, 16)][0]
        return (b * T + pj) * H, d * H

    def slab_copy(s):
        src, dst = rows(s)
        pltpu.async_copy(x_hbm.at[pl.ds(src, H)],
                         out_hbm.at[pl.ds(dst, H)], sems[s % NSEM])

    def slab_wait(s):
        src, dst = rows(s)
        pltpu.make_async_copy(x_hbm.at[pl.ds(src, H)],
                              out_hbm.at[pl.ds(dst, H)], sems[s % NSEM]).wait()

    for s in range(SPW):
        slab_copy(s)
    for s in range(SPW):
        slab_wait(s)


def kernel(x, idxs):
    xt = jnp.transpose(x, (0, 2, 3, 4, 1))
    xr = xt.reshape(ROWS, W, C)
    out = _sc_shuffle(xr, idxs.astype(jnp.int32))
    out5 = out.reshape(B, T, H, W, C)
    return jnp.transpose(out5, (0, 4, 1, 2, 3))


# speedup vs baseline: 40.6541x; 40.6541x over previous
import functools

import jax
import jax.numpy as jnp
from jax import lax
from jax.experimental import pallas as pl
from jax.experimental.pallas import tpu as pltpu, tpu_sc as plsc

B, C, T, H, W = 8, 64, 32, 56, 56
NC, NS = 2, 16
NW = NC * NS
SLABS = B * T
SPW = SLABS // NW
ROWS = SLABS * H
RCH = 8
CPS = H // RCH


@functools.partial(
    pl.kernel,
    out_type=jax.ShapeDtypeStruct((ROWS, W, C), jnp.float32),
    mesh=plsc.VectorSubcoreMesh(core_axis_name="c", subcore_axis_name="s"),
    scratch_types=[
        pltpu.VMEM((T + 16,), jnp.int32),
        pltpu.VMEM((RCH, W, C), jnp.float32),
        pltpu.VMEM((RCH, W, C), jnp.float32),
        pltpu.SemaphoreType.DMA,
        pltpu.SemaphoreType.DMA,
        pltpu.SemaphoreType.DMA,
        pltpu.SemaphoreType.DMA,
    ],
)
def _sc_shuffle(x_hbm, idx_hbm, out_hbm, idxs_v, buf0, buf1, g0, g1, o0, o1):
    wid = lax.axis_index("s") * NC + lax.axis_index("c")
    pltpu.sync_copy(idx_hbm, idxs_v.at[pl.ds(0, T)])

    bufs = (buf0, buf1)
    gsems = (g0, g1)
    osems = (o0, o1)

    def src_row(c):
        s, k = divmod(c, CPS)
        d = wid * SPW + s
        b = lax.shift_right_logical(d, 5)
        j = lax.bitwise_and(d, T - 1)
        pj = idxs_v[pl.ds(j, 16)][0]
        return (b * T + pj) * H + k * RCH

    def dst_row(c):
        s, k = divmod(c, CPS)
        return (wid * SPW + s) * H + k * RCH

    def gather(c):
        pltpu.async_copy(x_hbm.at[pl.ds(src_row(c), RCH)],
                         bufs[c % 2], gsems[c % 2])

    def gather_wait(c):
        pltpu.make_async_copy(x_hbm.at[pl.ds(src_row(c), RCH)],
                              bufs[c % 2], gsems[c % 2]).wait()

    def scatter(c):
        pltpu.async_copy(bufs[c % 2], out_hbm.at[pl.ds(dst_row(c), RCH)],
                         osems[c % 2])

    def scatter_wait(c):
        pltpu.make_async_copy(bufs[c % 2], out_hbm.at[pl.ds(dst_row(c), RCH)],
                              osems[c % 2]).wait()

    NCH = SPW * CPS
    gather(0)
    gather(1)
    for c in range(NCH):
        gather_wait(c)
        scatter(c)
        if c >= 1:
            scatter_wait(c - 1)
            if c + 1 < NCH:
                gather(c + 1)
    scatter_wait(NCH - 1)


def kernel(x, idxs):
    xt = jnp.transpose(x, (0, 2, 3, 4, 1))
    xr = xt.reshape(ROWS, W, C)
    out = _sc_shuffle(xr, idxs.astype(jnp.int32))
    out5 = out.reshape(B, T, H, W, C)
    return jnp.transpose(out5, (0, 4, 1, 2, 3))
